# baseline (speedup 1.0000x reference)
import functools

import jax
import jax.numpy as jnp
from jax import lax
from jax.experimental import pallas as pl
from jax.experimental.pallas import tpu as pltpu
from jax.experimental.pallas import tpu_sc as plsc

N_NODES = 10000
N_EDGES = 320000
D_FEAT = 128

_NC = 2
_NS = 16
_NW = _NC * _NS

_EPW = N_EDGES // _NW
_B = 128
_NB = _EPW // _B
_TAIL = _EPW - _NB * _B
_RPT = 624
_RPT_EXTRA = N_NODES - _NS * _RPT


def _sc_scatter_gather(x_hbm, src_hbm, dst_hbm, part_hbm,
                       sidx0, sidx1, sidx2, didx0, didx1, didx2,
                       rows0, rows1,
                       sidx_t, didx_t, rows_t, acc,
                       isem0, isem1, isem2, gsem0, gsem1, ssem0, ssem1, tsem):
    c = lax.axis_index("c")
    s = lax.axis_index("s")
    wid = s * _NC + c
    ebase = wid * _EPW

    sidx = (sidx0, sidx1, sidx2)
    didx = (didx0, didx1, didx2)
    rows = (rows0, rows1)
    isem = (isem0, isem1, isem2)
    gsem = (gsem0, gsem1)
    ssem = (ssem0, ssem1)

    def idx_start(j, i):
        off = ebase + j * _B
        pltpu.async_copy(src_hbm.at[pl.ds(off, _B)], sidx[i], isem[i])
        pltpu.async_copy(dst_hbm.at[pl.ds(off, _B)], didx[i], isem[i])

    def idx_wait(j, i):
        off = ebase + j * _B
        pltpu.make_async_copy(src_hbm.at[pl.ds(off, _B)], sidx[i], isem[i]).wait()
        pltpu.make_async_copy(dst_hbm.at[pl.ds(off, _B)], didx[i], isem[i]).wait()

    def gather_start(i, r):
        pltpu.async_copy(x_hbm.at[sidx[i]], rows[r], gsem[r])

    def gather_wait(i, r):
        pltpu.make_async_copy(x_hbm.at[sidx[i]], rows[r], gsem[r]).wait()

    def scatter_start(i, r):
        pltpu.async_copy(rows[r], acc.at[didx[i]], ssem[r], add=True)

    def scatter_wait(i, r):
        pltpu.make_async_copy(rows[r], acc.at[didx[i]], ssem[r]).wait()

    idx_start(0, 0)
    idx_start(1, 1)

    zero16 = jnp.zeros((16,), jnp.float32)
    def zrow(r, carry):
        for k in range(D_FEAT // 16):
            rows0[r, pl.ds(k * 16, 16)] = zero16
        return carry
    lax.fori_loop(0, _B, zrow, 0)
    z0 = s * _RPT
    for k in range(_RPT // _B):
        pltpu.sync_copy(rows0, acc.at[pl.ds(z0 + k * _B, _B)])
    rem = _RPT - (_RPT // _B) * _B
    if rem:
        pltpu.sync_copy(rows0.at[pl.ds(0, rem)],
                        acc.at[pl.ds(z0 + (_RPT // _B) * _B, rem)])

    @pl.when(s == _NS - 1)
    def _zero_extra():
        pltpu.sync_copy(rows0.at[pl.ds(0, _RPT_EXTRA)],
                        acc.at[pl.ds(_NS * _RPT, _RPT_EXTRA)])

    idx_wait(0, 0)
    gather_start(0, 0)
    plsc.subcore_barrier()

    def body(j, i3, r, first=0):
        i3n = (i3 + 1) % 3
        i3nn = (i3 + 2) % 3
        gather_wait(i3, r)
        scatter_start(i3, r)
        if first == 0:
            scatter_wait(i3nn, 1 - r)
        idx_start(j + 2, i3nn)
        idx_wait(j + 1, i3n)
        gather_start(i3n, 1 - r)

    body(0, 0, 0, first=1)
    body(1, 1, 1)
    body(2, 2, 0)
    body(3, 0, 1)

    def group(g, carry):
        for k in range(6):
            j = 4 + 6 * g + k
            body(j, (4 + k) % 3, k % 2)
        return carry
    lax.fori_loop(0, 12, group, 0)

    gather_wait(76 % 3, 0)
    scatter_start(76 % 3, 0)
    scatter_wait(75 % 3, 1)
    idx_wait(77, 77 % 3)
    gather_start(77 % 3, 1)
    gather_wait(77 % 3, 1)
    scatter_start(77 % 3, 1)
    scatter_wait(76 % 3, 0)
    scatter_wait(77 % 3, 1)

    if _TAIL:
        off = ebase + _NB * _B
        pltpu.sync_copy(src_hbm.at[pl.ds(off, _TAIL)], sidx_t)
        pltpu.sync_copy(dst_hbm.at[pl.ds(off, _TAIL)], didx_t)
        pltpu.async_copy(x_hbm.at[sidx_t], rows_t, tsem).wait()
        pltpu.sync_copy(rows_t, acc.at[didx_t], add=True)

    plsc.subcore_barrier()

    pltpu.sync_copy(acc.at[pl.ds(z0, _RPT)], part_hbm.at[c, pl.ds(z0, _RPT)])

    @pl.when(s == _NS - 1)
    def _write_extra():
        pltpu.sync_copy(acc.at[pl.ds(_NS * _RPT, _RPT_EXTRA)],
                        part_hbm.at[c, pl.ds(_NS * _RPT, _RPT_EXTRA)])


def _combine_body(p_ref, o_ref):
    o_ref[...] = p_ref[0] + p_ref[1]


def kernel(x, edge_index):
    assert x.shape == (N_NODES, D_FEAT)
    src = edge_index[0].astype(jnp.int32)
    dst = edge_index[1].astype(jnp.int32)

    mesh = plsc.VectorSubcoreMesh(core_axis_name="c", subcore_axis_name="s")
    sc_call = pl.kernel(
        _sc_scatter_gather,
        out_type=jax.ShapeDtypeStruct((_NC, N_NODES, D_FEAT), jnp.float32),
        mesh=mesh,
        scratch_types=(
            [pltpu.VMEM((_B,), jnp.int32)] * 6
            + [pltpu.VMEM((_B, D_FEAT), jnp.float32)] * 2
            + [pltpu.VMEM((_TAIL,), jnp.int32)] * 2
            + [pltpu.VMEM((_TAIL, D_FEAT), jnp.float32)]
            + [pltpu.VMEM_SHARED((N_NODES, D_FEAT), jnp.float32)]
            + [pltpu.SemaphoreType.DMA] * 8
        ),
    )
    partials = sc_call(x, src, dst)

    blk = 1000
    out = pl.pallas_call(
        _combine_body,
        out_shape=jax.ShapeDtypeStruct((N_NODES, D_FEAT), jnp.float32),
        grid=(N_NODES // blk,),
        in_specs=[pl.BlockSpec((_NC, blk, D_FEAT), lambda i: (0, i, 0))],
        out_specs=pl.BlockSpec((blk, D_FEAT), lambda i: (i, 0)),
    )(partials)
    return out


# speedup vs baseline: 1.0456x; 1.0456x over previous
import functools

import jax
import jax.numpy as jnp
from jax import lax
from jax.experimental import pallas as pl
from jax.experimental.pallas import tpu as pltpu
from jax.experimental.pallas import tpu_sc as plsc

N_NODES = 10000
N_EDGES = 320000
D_FEAT = 128

_NC = 2
_NS = 16
_NW = _NC * _NS

_EPW = N_EDGES // _NW
_B = 128
_NB = _EPW // _B
_TAIL = _EPW - _NB * _B
_RPT = 624
_RPT_EXTRA = N_NODES - _NS * _RPT


def _sc_scatter_gather(x_hbm, src_hbm, dst_hbm, part_hbm,
                       sidx0, sidx1, didx0, didx1, rows0, rows1,
                       sidx_t, didx_t, rows_t, acc,
                       gsem0, gsem1, isem0, isem1, tsem):
    c = lax.axis_index("c")
    s = lax.axis_index("s")
    wid = s * _NC + c
    ebase = wid * _EPW

    sidx = (sidx0, sidx1)
    didx = (didx0, didx1)
    rows = (rows0, rows1)
    gsem = (gsem0, gsem1)
    isem = (isem0, isem1)

    def idx_start(j, b):
        off = ebase + j * _B
        pltpu.async_copy(src_hbm.at[pl.ds(off, _B)], sidx[b], isem[b])
        pltpu.async_copy(dst_hbm.at[pl.ds(off, _B)], didx[b], isem[b])

    def idx_wait(j, b):
        off = ebase + j * _B
        pltpu.make_async_copy(src_hbm.at[pl.ds(off, _B)], sidx[b], isem[b]).wait()
        pltpu.make_async_copy(dst_hbm.at[pl.ds(off, _B)], didx[b], isem[b]).wait()

    def gather_start(b):
        pltpu.async_copy(x_hbm.at[sidx[b]], rows[b], gsem[b])

    def gather_wait(b):
        pltpu.make_async_copy(x_hbm.at[sidx[b]], rows[b], gsem[b]).wait()

    def scatter(b):
        pltpu.sync_copy(rows[b], acc.at[didx[b]], add=True)

    idx_start(0, 0)
    idx_start(1, 1)

    zero16 = jnp.zeros((16,), jnp.float32)
    def zrow(r, carry):
        for k in range(D_FEAT // 16):
            rows0[r, pl.ds(k * 16, 16)] = zero16
        return carry
    lax.fori_loop(0, _B, zrow, 0)
    z0 = s * _RPT
    for k in range(_RPT // _B):
        pltpu.sync_copy(rows0, acc.at[pl.ds(z0 + k * _B, _B)])
    rem = _RPT - (_RPT // _B) * _B
    if rem:
        pltpu.sync_copy(rows0.at[pl.ds(0, rem)],
                        acc.at[pl.ds(z0 + (_RPT // _B) * _B, rem)])

    @pl.when(s == _NS - 1)
    def _zero_extra():
        pltpu.sync_copy(rows0.at[pl.ds(0, _RPT_EXTRA)],
                        acc.at[pl.ds(_NS * _RPT, _RPT_EXTRA)])

    idx_wait(0, 0)
    gather_start(0)
    plsc.subcore_barrier()

    def group(g, carry):
        for b in range(2):
            j = 2 * g + b
            idx_wait(j + 1, 1 - b)
            gather_start(1 - b)
            gather_wait(b)
            scatter(b)
            idx_start(j + 2, b)
        return carry
    lax.fori_loop(0, (_NB - 2) // 2, group, 0)

    idx_wait(_NB - 1, 1)
    gather_start(1)
    gather_wait(0)
    scatter(0)
    gather_wait(1)
    scatter(1)

    if _TAIL:
        off = ebase + _NB * _B
        pltpu.sync_copy(src_hbm.at[pl.ds(off, _TAIL)], sidx_t)
        pltpu.sync_copy(dst_hbm.at[pl.ds(off, _TAIL)], didx_t)
        pltpu.async_copy(x_hbm.at[sidx_t], rows_t, tsem).wait()
        pltpu.sync_copy(rows_t, acc.at[didx_t], add=True)

    plsc.subcore_barrier()

    pltpu.sync_copy(acc.at[pl.ds(z0, _RPT)], part_hbm.at[c, pl.ds(z0, _RPT)])

    @pl.when(s == _NS - 1)
    def _write_extra():
        pltpu.sync_copy(acc.at[pl.ds(_NS * _RPT, _RPT_EXTRA)],
                        part_hbm.at[c, pl.ds(_NS * _RPT, _RPT_EXTRA)])


def _combine_body(p_ref, o_ref):
    o_ref[...] = p_ref[0] + p_ref[1]


def kernel(x, edge_index):
    assert x.shape == (N_NODES, D_FEAT)
    src = edge_index[0].astype(jnp.int32)
    dst = edge_index[1].astype(jnp.int32)

    mesh = plsc.VectorSubcoreMesh(core_axis_name="c", subcore_axis_name="s")
    sc_call = pl.kernel(
        _sc_scatter_gather,
        out_type=jax.ShapeDtypeStruct((_NC, N_NODES, D_FEAT), jnp.float32),
        mesh=mesh,
        scratch_types=[
            pltpu.VMEM((_B,), jnp.int32),
            pltpu.VMEM((_B,), jnp.int32),
            pltpu.VMEM((_B,), jnp.int32),
            pltpu.VMEM((_B,), jnp.int32),
            pltpu.VMEM((_B, D_FEAT), jnp.float32),
            pltpu.VMEM((_B, D_FEAT), jnp.float32),
            pltpu.VMEM((_TAIL,), jnp.int32),
            pltpu.VMEM((_TAIL,), jnp.int32),
            pltpu.VMEM((_TAIL, D_FEAT), jnp.float32),
            pltpu.VMEM_SHARED((N_NODES, D_FEAT), jnp.float32),
            pltpu.SemaphoreType.DMA,
            pltpu.SemaphoreType.DMA,
            pltpu.SemaphoreType.DMA,
            pltpu.SemaphoreType.DMA,
            pltpu.SemaphoreType.DMA,
        ],
    )
    partials = sc_call(x, src, dst)

    blk = 1000
    out = pl.pallas_call(
        _combine_body,
        out_shape=jax.ShapeDtypeStruct((N_NODES, D_FEAT), jnp.float32),
        grid=(N_NODES // blk,),
        in_specs=[pl.BlockSpec((_NC, blk, D_FEAT), lambda i: (0, i, 0))],
        out_specs=pl.BlockSpec((blk, D_FEAT), lambda i: (i, 0)),
    )(partials)
    return out
